# Initial kernel scaffold; baseline (speedup 1.0000x reference)
import jax
import jax.numpy as jnp
from jax.experimental import pallas as pl


def kernel(x, gi0, gi1, fgc0, fgc1, WM00, bM00, WU00, bU00, WM01, bM01, WU01, bU01, WM10, bM10, WU10, bU10, WM11, bM11, WU11, bU11):
    raise NotImplementedError("write your pallas kernel here")


# speedup vs baseline: 1.1924x; 1.1924x over previous
import functools

import jax
import jax.numpy as jnp
from jax import lax
from jax.experimental import pallas as pl
from jax.experimental.pallas import tpu as pltpu

N_ATOMS = 50000
D = 64
EMB = 64
G = 200000
NEG = float("-inf")


def _mm_kernel(a_ref, w_ref, b_ref, o_ref):
    o_ref[...] = jnp.dot(a_ref[...], w_ref[...],
                         preferred_element_type=jnp.float32) + b_ref[...]


def _mm(a, w, b, bm=2000):
    m = a.shape[0]
    grid = (m // bm,)
    return pl.pallas_call(
        _mm_kernel,
        grid=grid,
        in_specs=[pl.BlockSpec((bm, a.shape[1]), lambda i: (i, 0)),
                  pl.BlockSpec((a.shape[1], w.shape[1]), lambda i: (0, 0)),
                  pl.BlockSpec((1, w.shape[1]), lambda i: (0, 0))],
        out_specs=pl.BlockSpec((bm, w.shape[1]), lambda i: (i, 0)),
        out_shape=jax.ShapeDtypeStruct((m, w.shape[1]), jnp.float32),
    )(a, w, b[None])


def _combine_kernel(u0_ref, s0_ref, u1_ref, s1_ref, o_ref):
    r0 = jnp.maximum(u0_ref[...] + s0_ref[...], 0.0)
    r1 = jnp.maximum(u1_ref[...] + s1_ref[...], 0.0)
    o_ref[...] = jnp.maximum(r0, r1)


def _combine(u0, s0, u1, s1, bm=2000):
    m = u0.shape[0]
    spec = pl.BlockSpec((bm, u0.shape[1]), lambda i: (i, 0))
    return pl.pallas_call(
        _combine_kernel,
        grid=(m // bm,),
        in_specs=[spec] * 4,
        out_specs=spec,
        out_shape=jax.ShapeDtypeStruct(u0.shape, jnp.float32),
    )(u0, s0, u1, s1)


def _relu_add_kernel(a_ref, b_ref, o_ref):
    o_ref[...] = jnp.maximum(a_ref[...] + b_ref[...], 0.0)


def _relu_add(a, b, bm=2000):
    m = a.shape[0]
    spec = pl.BlockSpec((bm, a.shape[1]), lambda i: (i, 0))
    return pl.pallas_call(
        _relu_add_kernel,
        grid=(m // bm,),
        in_specs=[spec, spec],
        out_specs=spec,
        out_shape=jax.ShapeDtypeStruct(a.shape, jnp.float32),
    )(a, b)


def _gather_max(table, gi):
    return jnp.max(table[gi], axis=1)


def _scatter_max(c, gi, n_atoms):
    arity = gi.shape[1]
    smax = jnp.full((n_atoms, EMB), NEG, jnp.float32)
    vals = jnp.repeat(c, arity, axis=0)
    return smax.at[gi.reshape(-1)].max(vals, mode="drop")


def kernel(x, gi0, gi1, fgc0, fgc1,
           WM00, bM00, WU00, bU00, WM01, bM01, WU01, bU01,
           WM10, bM10, WU10, bU10, WM11, bM11, WU11, bU11):
    atom = x[0]
    pad = 0
    aU0 = _mm(atom, WU00[:D], bU00 * 0)
    aU1 = _mm(atom, WU10[:D], bU10 * 0)
    aM0 = _mm(atom, WM00[:D], bM00 * 0)
    aM1 = _mm(atom, WM10[:D], bM10 * 0)
    cU0 = _mm(fgc0, WU00[D:], bU00)
    cU1 = _mm(fgc1, WU10[D:], bU10)
    cM0 = _mm(fgc0, WM00[D:], bM00)
    cM1 = _mm(fgc1, WM10[D:], bM10)

    last0 = _relu_add(_gather_max(aM0, gi0), cM0)
    last1 = _relu_add(_gather_max(aM1, gi1), cM1)

    s0 = _scatter_max(cU0, gi0, N_ATOMS)
    s1 = _scatter_max(cU1, gi1, N_ATOMS)
    atom1 = _combine(aU0, s0, aU1, s1)

    aU0b = _mm(atom1, WU01[:D], bU01 * 0)
    aU1b = _mm(atom1, WU11[:D], bU11 * 0)
    cU0b = _mm(last0, WU01[D:], bU01)
    cU1b = _mm(last1, WU11[D:], bU11)
    s0b = _scatter_max(cU0b, gi0, N_ATOMS)
    s1b = _scatter_max(cU1b, gi1, N_ATOMS)
    out = _combine(aU0b, s0b, aU1b, s1b)
    return out[None]
